# Initial kernel scaffold; baseline (speedup 1.0000x reference)
import jax
import jax.numpy as jnp
from jax.experimental import pallas as pl


def kernel(x, edge_index, batch, W1, b1, g1, beta1, W2, b2, g2, beta2, W3, b3, g3, beta3, lin1_W, lin1_b, lin2_W, lin2_b):
    raise NotImplementedError("write your pallas kernel here")


# speedup vs baseline: 7.0133x; 7.0133x over previous
import functools

import jax
import jax.numpy as jnp
from jax import lax
from jax.experimental import pallas as pl
from jax.experimental.pallas import tpu as pltpu
from jax.experimental.pallas import tpu_sc as plsc

N = 10000
NPAD = 10240
E = 320000
D = 128
G = 128
C = 10

NC = 2
NS = 16
CHUNK = 128
EPAD = 327680
CHUNKS_PER_TILE = EPAD // (NC * NS * CHUNK)
ROWS_PER_TILE = NPAD // NS
DEGW = 128

def _copy_idx_row(src2d, j, dst1d):
    for t in range(CHUNK // 16):
        dst1d[pl.ds(t * 16, 16)] = src2d[j, pl.ds(t * 16, 16)]


def _deg_sc_body(dst_hbm, ones_hbm, zeros_hbm, out_hbm, dst_v, ones_v, idx1,
                 accum):
    c = lax.axis_index("c")
    s = lax.axis_index("s")
    wid = s * NC + c
    pltpu.sync_copy(zeros_hbm, accum.at[pl.ds(s * ROWS_PER_TILE, ROWS_PER_TILE)])
    pltpu.sync_copy(ones_hbm, ones_v)
    pltpu.sync_copy(dst_hbm.at[pl.ds(wid * CHUNKS_PER_TILE, CHUNKS_PER_TILE)], dst_v)
    plsc.subcore_barrier()

    def body(j, carry):
        _copy_idx_row(dst_v, j, idx1)
        pltpu.sync_copy(ones_v, accum.at[idx1], add=True)
        return carry

    lax.fori_loop(0, CHUNKS_PER_TILE, body, 0)
    plsc.subcore_barrier()
    pltpu.sync_copy(
        accum.at[pl.ds(s * ROWS_PER_TILE, ROWS_PER_TILE)],
        out_hbm.at[c, pl.ds(s * ROWS_PER_TILE, ROWS_PER_TILE)],
    )


def _agg_sc_body(y_hbm, src_hbm, dst_hbm, zeros_hbm, out_hbm, src_v, dst_v,
                 rows_v, idx0, idx1, accum, sem):
    c = lax.axis_index("c")
    s = lax.axis_index("s")
    wid = s * NC + c
    pltpu.sync_copy(zeros_hbm, accum.at[pl.ds(s * ROWS_PER_TILE, ROWS_PER_TILE)])
    pltpu.sync_copy(src_hbm.at[pl.ds(wid * CHUNKS_PER_TILE, CHUNKS_PER_TILE)], src_v)
    pltpu.sync_copy(dst_hbm.at[pl.ds(wid * CHUNKS_PER_TILE, CHUNKS_PER_TILE)], dst_v)
    plsc.subcore_barrier()

    def body(j, carry):
        _copy_idx_row(src_v, j, idx0)
        _copy_idx_row(dst_v, j, idx1)
        pltpu.async_copy(y_hbm.at[idx0], rows_v, sem).wait()
        pltpu.sync_copy(rows_v, accum.at[idx1], add=True)
        return carry

    lax.fori_loop(0, CHUNKS_PER_TILE, body, 0)
    plsc.subcore_barrier()
    pltpu.sync_copy(
        accum.at[pl.ds(s * ROWS_PER_TILE, ROWS_PER_TILE)],
        out_hbm.at[c, pl.ds(s * ROWS_PER_TILE, ROWS_PER_TILE)],
    )


@functools.cache
def _sc_kernels():
    mesh = plsc.VectorSubcoreMesh(
        core_axis_name="c", subcore_axis_name="s", num_cores=NC, num_subcores=NS
    )
    deg = pl.kernel(
        _deg_sc_body,
        out_type=jax.ShapeDtypeStruct((NC, NPAD, DEGW), jnp.float32),
        mesh=mesh,
        scratch_types=[
            pltpu.VMEM((CHUNKS_PER_TILE, CHUNK), jnp.int32),
            pltpu.VMEM((CHUNK, DEGW), jnp.float32),
            pltpu.VMEM((CHUNK,), jnp.int32),
            pltpu.VMEM_SHARED((NPAD, DEGW), jnp.float32),
        ],
    )
    agg = pl.kernel(
        _agg_sc_body,
        out_type=jax.ShapeDtypeStruct((NC, NPAD, D), jnp.float32),
        mesh=mesh,
        scratch_types=[
            pltpu.VMEM((CHUNKS_PER_TILE, CHUNK), jnp.int32),
            pltpu.VMEM((CHUNKS_PER_TILE, CHUNK), jnp.int32),
            pltpu.VMEM((CHUNK, D), jnp.float32),
            pltpu.VMEM((CHUNK,), jnp.int32),
            pltpu.VMEM((CHUNK,), jnp.int32),
            pltpu.VMEM_SHARED((NPAD, D), jnp.float32),
            pltpu.SemaphoreType.DMA,
        ],
    )
    return deg, agg


def _prep_body(deg_ref, x_ref, w_ref, dinv_ref, y_ref):
    deg = deg_ref[0, :, 0:1] + deg_ref[1, :, 0:1] + 1.0
    dinv = lax.rsqrt(deg)
    dinv_ref[...] = dinv
    y_ref[...] = (
        jnp.dot(x_ref[...], w_ref[...], preferred_element_type=jnp.float32) * dinv
    )


def _mid_layer(p_ref, y_ref, dinv_ref, b_ref, g_ref, beta_ref):
    pre = (p_ref[0] + p_ref[1] + y_ref[...]) * dinv_ref[...] + b_ref[...]
    valid = lax.broadcasted_iota(jnp.int32, (NPAD, 1), 0) < N
    pre = jnp.where(valid, pre, 0.0)
    mu = jnp.sum(pre, axis=0, keepdims=True) * (1.0 / N)
    var = jnp.sum(pre * pre, axis=0, keepdims=True) * (1.0 / N) - mu * mu
    h = (pre - mu) * lax.rsqrt(var + 1e-5) * g_ref[...] + beta_ref[...]
    h = jnp.maximum(h, 0.0)
    return jnp.where(valid, h, 0.0), valid


def _bn_body(p_ref, y_ref, dinv_ref, b_ref, g_ref, beta_ref, w_ref, ynext_ref):
    h, _ = _mid_layer(p_ref, y_ref, dinv_ref, b_ref, g_ref, beta_ref)
    ynext_ref[...] = (
        jnp.dot(h, w_ref[...], preferred_element_type=jnp.float32) * dinv_ref[...]
    )


def _head_body(p_ref, y_ref, dinv_ref, b_ref, g_ref, beta_ref, batch_ref,
               l1w_ref, l1b_ref, l2w_ref, l2b_ref, out_ref):
    h, valid = _mid_layer(p_ref, y_ref, dinv_ref, b_ref, g_ref, beta_ref)
    onehot = (
        batch_ref[...] == lax.broadcasted_iota(jnp.int32, (NPAD, G), 1)
    ).astype(jnp.float32)
    validf = valid.astype(jnp.float32)
    onehot = onehot * validf
    sums = lax.dot_general(
        onehot, h, (((0,), (0,)), ((), ())), preferred_element_type=jnp.float32
    )
    cnts = lax.dot_general(
        onehot, validf, (((0,), (0,)), ((), ())), preferred_element_type=jnp.float32
    )
    pooled = sums / jnp.maximum(cnts, 1.0)
    o = jnp.dot(pooled, l1w_ref[...], preferred_element_type=jnp.float32) + l1b_ref[...]
    o = jnp.maximum(o, 0.0)
    o = jnp.dot(o, l2w_ref[...], preferred_element_type=jnp.float32) + l2b_ref[...]
    m = jnp.max(o, axis=1, keepdims=True)
    lse = jnp.log(jnp.sum(jnp.exp(o - m), axis=1, keepdims=True)) + m
    out_ref[...] = o - lse


_prep_call = pl.pallas_call(
    _prep_body,
    out_shape=[
        jax.ShapeDtypeStruct((NPAD, 1), jnp.float32),
        jax.ShapeDtypeStruct((NPAD, D), jnp.float32),
    ],
)

_bn_call = pl.pallas_call(
    _bn_body,
    out_shape=jax.ShapeDtypeStruct((NPAD, D), jnp.float32),
)

_head_call = pl.pallas_call(
    _head_body,
    out_shape=jax.ShapeDtypeStruct((G, C), jnp.float32),
)


def kernel(x, edge_index, batch, W1, b1, g1, beta1, W2, b2, g2, beta2,
           W3, b3, g3, beta3, lin1_W, lin1_b, lin2_W, lin2_b):
    f32 = jnp.float32
    pad_e = EPAD - E
    src = jnp.concatenate(
        [edge_index[0], jnp.full((pad_e,), NPAD - 1, jnp.int32)]
    ).reshape(EPAD // CHUNK, CHUNK)
    dst = jnp.concatenate(
        [edge_index[1], jnp.full((pad_e,), NPAD - 1, jnp.int32)]
    ).reshape(EPAD // CHUNK, CHUNK)
    x_pad = jnp.zeros((NPAD, D), f32).at[:N].set(x)
    batch_pad = jnp.zeros((NPAD, 1), jnp.int32).at[:N, 0].set(batch)

    ones_rows = jnp.ones((CHUNK, DEGW), f32)
    zeros_deg = jnp.zeros((ROWS_PER_TILE, DEGW), f32)
    zeros_agg = jnp.zeros((ROWS_PER_TILE, D), f32)

    b1r, g1r, be1 = b1.reshape(1, D), g1.reshape(1, D), beta1.reshape(1, D)
    b2r, g2r, be2 = b2.reshape(1, D), g2.reshape(1, D), beta2.reshape(1, D)
    b3r, g3r, be3 = b3.reshape(1, D), g3.reshape(1, D), beta3.reshape(1, D)
    l1b = lin1_b.reshape(1, D)
    l2b = lin2_b.reshape(1, C)

    _deg_sc, _agg_sc = _sc_kernels()
    deg_parts = _deg_sc(dst, ones_rows, zeros_deg)
    dinv, y1 = _prep_call(deg_parts, x_pad, W1)

    p1 = _agg_sc(y1, src, dst, zeros_agg)
    y2 = _bn_call(p1, y1, dinv, b1r, g1r, be1, W2)

    p2 = _agg_sc(y2, src, dst, zeros_agg)
    y3 = _bn_call(p2, y2, dinv, b2r, g2r, be2, W3)

    p3 = _agg_sc(y3, src, dst, zeros_agg)
    out = _head_call(p3, y3, dinv, b3r, g3r, be3, batch_pad,
                     lin1_W, l1b, lin2_W, l2b)
    return out


# speedup vs baseline: 7.8583x; 1.1205x over previous
import functools

import jax
import jax.numpy as jnp
from jax import lax
from jax.experimental import pallas as pl
from jax.experimental.pallas import tpu as pltpu
from jax.experimental.pallas import tpu_sc as plsc

N = 10000
NPAD = 10240
E = 320000
D = 128
G = 128
C = 10

NC = 2
NS = 16
CHUNK = 128
EPAD = 327680
CHUNKS_PER_TILE = EPAD // (NC * NS * CHUNK)
IDXBLK = 40
ROWS_PER_TILE = NPAD // NS
DEGW = 128

def _copy_idx_row(src2d, j, dst1d):
    for t in range(CHUNK // 16):
        dst1d[pl.ds(t * 16, 16)] = src2d[j, pl.ds(t * 16, 16)]


def _deg_sc_body(dst_hbm, ones_hbm, zeros_hbm, out_hbm, dst_v, ones_v, idx1,
                 accum):
    c = lax.axis_index("c")
    s = lax.axis_index("s")
    wid = s * NC + c
    pltpu.sync_copy(zeros_hbm, accum.at[pl.ds(s * ROWS_PER_TILE, ROWS_PER_TILE)])
    pltpu.sync_copy(ones_hbm, ones_v)
    pltpu.sync_copy(dst_hbm.at[pl.ds(wid * CHUNKS_PER_TILE, CHUNKS_PER_TILE)], dst_v)
    plsc.subcore_barrier()

    def body(j, carry):
        _copy_idx_row(dst_v, j, idx1)
        pltpu.sync_copy(ones_v, accum.at[idx1], add=True)
        return carry

    lax.fori_loop(0, CHUNKS_PER_TILE, body, 0)
    plsc.subcore_barrier()
    pltpu.sync_copy(
        accum.at[pl.ds(s * ROWS_PER_TILE, ROWS_PER_TILE)],
        out_hbm.at[c, pl.ds(s * ROWS_PER_TILE, ROWS_PER_TILE)],
    )


def _agg_sc_body(y_hbm, src_hbm, dst_hbm, zeros_hbm, out_hbm, src_v, dst_v,
                 rows_a, rows_b, ia, ib, idx1, accum, sema, semb):
    c = lax.axis_index("c")
    s = lax.axis_index("s")
    wid = s * NC + c
    pltpu.sync_copy(zeros_hbm, accum.at[pl.ds(s * ROWS_PER_TILE, ROWS_PER_TILE)])
    plsc.subcore_barrier()

    for blk in range(CHUNKS_PER_TILE // IDXBLK):
        base = wid * CHUNKS_PER_TILE + blk * IDXBLK
        pltpu.sync_copy(src_hbm.at[pl.ds(base, IDXBLK)], src_v)
        pltpu.sync_copy(dst_hbm.at[pl.ds(base, IDXBLK)], dst_v)
        _copy_idx_row(src_v, 0, ia)
        pltpu.async_copy(y_hbm.at[ia], rows_a, sema)

        def pair(k, carry):
            j0 = 2 * k
            _copy_idx_row(src_v, j0 + 1, ib)
            pltpu.async_copy(y_hbm.at[ib], rows_b, semb)
            pltpu.make_async_copy(y_hbm.at[ia], rows_a, sema).wait()
            _copy_idx_row(dst_v, j0, idx1)
            pltpu.sync_copy(rows_a, accum.at[idx1], add=True)

            @pl.when(k < IDXBLK // 2 - 1)
            def _():
                _copy_idx_row(src_v, j0 + 2, ia)
                pltpu.async_copy(y_hbm.at[ia], rows_a, sema)

            pltpu.make_async_copy(y_hbm.at[ib], rows_b, semb).wait()
            _copy_idx_row(dst_v, j0 + 1, idx1)
            pltpu.sync_copy(rows_b, accum.at[idx1], add=True)
            return carry

        lax.fori_loop(0, IDXBLK // 2, pair, 0)
    plsc.subcore_barrier()
    pltpu.sync_copy(
        accum.at[pl.ds(s * ROWS_PER_TILE, ROWS_PER_TILE)],
        out_hbm.at[c, pl.ds(s * ROWS_PER_TILE, ROWS_PER_TILE)],
    )


@functools.cache
def _sc_kernels():
    mesh = plsc.VectorSubcoreMesh(
        core_axis_name="c", subcore_axis_name="s", num_cores=NC, num_subcores=NS
    )
    deg = pl.kernel(
        _deg_sc_body,
        out_type=jax.ShapeDtypeStruct((NC, NPAD, DEGW), jnp.float32),
        mesh=mesh,
        scratch_types=[
            pltpu.VMEM((CHUNKS_PER_TILE, CHUNK), jnp.int32),
            pltpu.VMEM((CHUNK, DEGW), jnp.float32),
            pltpu.VMEM((CHUNK,), jnp.int32),
            pltpu.VMEM_SHARED((NPAD, DEGW), jnp.float32),
        ],
    )
    agg = pl.kernel(
        _agg_sc_body,
        out_type=jax.ShapeDtypeStruct((NC, NPAD, D), jnp.float32),
        mesh=mesh,
        scratch_types=[
            pltpu.VMEM((IDXBLK, CHUNK), jnp.int32),
            pltpu.VMEM((IDXBLK, CHUNK), jnp.int32),
            pltpu.VMEM((CHUNK, D), jnp.float32),
            pltpu.VMEM((CHUNK, D), jnp.float32),
            pltpu.VMEM((CHUNK,), jnp.int32),
            pltpu.VMEM((CHUNK,), jnp.int32),
            pltpu.VMEM((CHUNK,), jnp.int32),
            pltpu.VMEM_SHARED((NPAD, D), jnp.float32),
            pltpu.SemaphoreType.DMA,
            pltpu.SemaphoreType.DMA,
        ],
    )
    return deg, agg


def _prep_body(deg_ref, x_ref, w_ref, dinv_ref, y_ref):
    deg = deg_ref[0, :, 0:1] + deg_ref[1, :, 0:1] + 1.0
    dinv = lax.rsqrt(deg)
    dinv_ref[...] = dinv
    y_ref[...] = (
        jnp.dot(x_ref[...], w_ref[...], preferred_element_type=jnp.float32) * dinv
    )


def _mid_layer(p_ref, y_ref, dinv_ref, b_ref, g_ref, beta_ref):
    pre = (p_ref[0] + p_ref[1] + y_ref[...]) * dinv_ref[...] + b_ref[...]
    valid = lax.broadcasted_iota(jnp.int32, (NPAD, 1), 0) < N
    pre = jnp.where(valid, pre, 0.0)
    mu = jnp.sum(pre, axis=0, keepdims=True) * (1.0 / N)
    var = jnp.sum(pre * pre, axis=0, keepdims=True) * (1.0 / N) - mu * mu
    h = (pre - mu) * lax.rsqrt(var + 1e-5) * g_ref[...] + beta_ref[...]
    h = jnp.maximum(h, 0.0)
    return jnp.where(valid, h, 0.0), valid


def _bn_body(p_ref, y_ref, dinv_ref, b_ref, g_ref, beta_ref, w_ref, ynext_ref):
    h, _ = _mid_layer(p_ref, y_ref, dinv_ref, b_ref, g_ref, beta_ref)
    ynext_ref[...] = (
        jnp.dot(h, w_ref[...], preferred_element_type=jnp.float32) * dinv_ref[...]
    )


def _head_body(p_ref, y_ref, dinv_ref, b_ref, g_ref, beta_ref, batch_ref,
               l1w_ref, l1b_ref, l2w_ref, l2b_ref, out_ref):
    h, valid = _mid_layer(p_ref, y_ref, dinv_ref, b_ref, g_ref, beta_ref)
    onehot = (
        batch_ref[...] == lax.broadcasted_iota(jnp.int32, (NPAD, G), 1)
    ).astype(jnp.float32)
    validf = valid.astype(jnp.float32)
    onehot = onehot * validf
    sums = lax.dot_general(
        onehot, h, (((0,), (0,)), ((), ())), preferred_element_type=jnp.float32
    )
    cnts = lax.dot_general(
        onehot, validf, (((0,), (0,)), ((), ())), preferred_element_type=jnp.float32
    )
    pooled = sums / jnp.maximum(cnts, 1.0)
    o = jnp.dot(pooled, l1w_ref[...], preferred_element_type=jnp.float32) + l1b_ref[...]
    o = jnp.maximum(o, 0.0)
    o = jnp.dot(o, l2w_ref[...], preferred_element_type=jnp.float32) + l2b_ref[...]
    m = jnp.max(o, axis=1, keepdims=True)
    lse = jnp.log(jnp.sum(jnp.exp(o - m), axis=1, keepdims=True)) + m
    out_ref[...] = o - lse


_prep_call = pl.pallas_call(
    _prep_body,
    out_shape=[
        jax.ShapeDtypeStruct((NPAD, 1), jnp.float32),
        jax.ShapeDtypeStruct((NPAD, D), jnp.float32),
    ],
)

_bn_call = pl.pallas_call(
    _bn_body,
    out_shape=jax.ShapeDtypeStruct((NPAD, D), jnp.float32),
)

_head_call = pl.pallas_call(
    _head_body,
    out_shape=jax.ShapeDtypeStruct((G, C), jnp.float32),
)


def kernel(x, edge_index, batch, W1, b1, g1, beta1, W2, b2, g2, beta2,
           W3, b3, g3, beta3, lin1_W, lin1_b, lin2_W, lin2_b):
    f32 = jnp.float32
    pad_e = EPAD - E
    src = jnp.concatenate(
        [edge_index[0], jnp.full((pad_e,), NPAD - 1, jnp.int32)]
    ).reshape(EPAD // CHUNK, CHUNK)
    dst = jnp.concatenate(
        [edge_index[1], jnp.full((pad_e,), NPAD - 1, jnp.int32)]
    ).reshape(EPAD // CHUNK, CHUNK)
    x_pad = jnp.zeros((NPAD, D), f32).at[:N].set(x)
    batch_pad = jnp.zeros((NPAD, 1), jnp.int32).at[:N, 0].set(batch)

    ones_rows = jnp.ones((CHUNK, DEGW), f32)
    zeros_deg = jnp.zeros((ROWS_PER_TILE, DEGW), f32)
    zeros_agg = jnp.zeros((ROWS_PER_TILE, D), f32)

    b1r, g1r, be1 = b1.reshape(1, D), g1.reshape(1, D), beta1.reshape(1, D)
    b2r, g2r, be2 = b2.reshape(1, D), g2.reshape(1, D), beta2.reshape(1, D)
    b3r, g3r, be3 = b3.reshape(1, D), g3.reshape(1, D), beta3.reshape(1, D)
    l1b = lin1_b.reshape(1, D)
    l2b = lin2_b.reshape(1, C)

    _deg_sc, _agg_sc = _sc_kernels()
    deg_parts = _deg_sc(dst, ones_rows, zeros_deg)
    dinv, y1 = _prep_call(deg_parts, x_pad, W1)

    p1 = _agg_sc(y1, src, dst, zeros_agg)
    y2 = _bn_call(p1, y1, dinv, b1r, g1r, be1, W2)

    p2 = _agg_sc(y2, src, dst, zeros_agg)
    y3 = _bn_call(p2, y2, dinv, b2r, g2r, be2, W3)

    p3 = _agg_sc(y3, src, dst, zeros_agg)
    out = _head_call(p3, y3, dinv, b3r, g3r, be3, batch_pad,
                     lin1_W, l1b, lin2_W, l2b)
    return out


# speedup vs baseline: 8.2980x; 1.0560x over previous
import functools

import jax
import jax.numpy as jnp
from jax import lax
from jax.experimental import pallas as pl
from jax.experimental.pallas import tpu as pltpu
from jax.experimental.pallas import tpu_sc as plsc

N = 10000
NPAD = 10240
E = 320000
D = 128
G = 128
C = 10

NC = 2
NS = 16
CHUNK = 128
EPAD = 327680
CHUNKS_PER_TILE = EPAD // (NC * NS * CHUNK)
IDXBLK = 40
CPT0 = 120
CPT1 = 40
ROWS_PER_TILE = NPAD // NS
DEGW = 128

def _copy_idx_row(src2d, j, dst1d):
    for t in range(CHUNK // 16):
        dst1d[pl.ds(t * 16, 16)] = src2d[j, pl.ds(t * 16, 16)]


def _deg_sc_body(dst_hbm, ones_hbm, zeros_hbm, out_hbm, dst_v, ones_v, idx1,
                 accum):
    c = lax.axis_index("c")
    s = lax.axis_index("s")
    wid = s * NC + c
    pltpu.sync_copy(zeros_hbm, accum.at[pl.ds(s * ROWS_PER_TILE, ROWS_PER_TILE)])
    pltpu.sync_copy(ones_hbm, ones_v)
    pltpu.sync_copy(dst_hbm.at[pl.ds(wid * CHUNKS_PER_TILE, CHUNKS_PER_TILE)], dst_v)
    plsc.subcore_barrier()

    def body(j, carry):
        _copy_idx_row(dst_v, j, idx1)
        pltpu.sync_copy(ones_v, accum.at[idx1], add=True)
        return carry

    lax.fori_loop(0, CHUNKS_PER_TILE, body, 0)
    plsc.subcore_barrier()
    pltpu.sync_copy(
        accum.at[pl.ds(s * ROWS_PER_TILE, ROWS_PER_TILE)],
        out_hbm.at[c, pl.ds(s * ROWS_PER_TILE, ROWS_PER_TILE)],
    )


def _agg_sc_body(y_hbm, src_hbm, dst_hbm, zeros_hbm, out_hbm, src_v, dst_v,
                 rows_a, rows_b, ia, ib, idx1, accum, sema, semb):
    c = lax.axis_index("c")
    s = lax.axis_index("s")
    wid = s * NC + c
    pltpu.sync_copy(zeros_hbm, accum.at[pl.ds(s * ROWS_PER_TILE, ROWS_PER_TILE)])
    plsc.subcore_barrier()

    nblk = jnp.where(c == 0, CPT0 // IDXBLK, CPT1 // IDXBLK)
    tile_base = jnp.where(
        c == 0, s * CPT0, NS * CPT0 + s * CPT1
    )
    for blk in range(CPT0 // IDXBLK):
        @pl.when(blk < nblk)
        def _():
            base = tile_base + blk * IDXBLK
            pltpu.sync_copy(src_hbm.at[pl.ds(base, IDXBLK)], src_v)
            pltpu.sync_copy(dst_hbm.at[pl.ds(base, IDXBLK)], dst_v)
            _copy_idx_row(src_v, 0, ia)
            pltpu.async_copy(y_hbm.at[ia], rows_a, sema)

            def pair(k, carry):
                j0 = 2 * k
                _copy_idx_row(src_v, j0 + 1, ib)
                pltpu.async_copy(y_hbm.at[ib], rows_b, semb)
                pltpu.make_async_copy(y_hbm.at[ia], rows_a, sema).wait()
                _copy_idx_row(dst_v, j0, idx1)
                pltpu.sync_copy(rows_a, accum.at[idx1], add=True)

                @pl.when(k < IDXBLK // 2 - 1)
                def _():
                    _copy_idx_row(src_v, j0 + 2, ia)
                    pltpu.async_copy(y_hbm.at[ia], rows_a, sema)

                pltpu.make_async_copy(y_hbm.at[ib], rows_b, semb).wait()
                _copy_idx_row(dst_v, j0 + 1, idx1)
                pltpu.sync_copy(rows_b, accum.at[idx1], add=True)
                return carry

            lax.fori_loop(0, IDXBLK // 2, pair, 0)
    plsc.subcore_barrier()
    pltpu.sync_copy(
        accum.at[pl.ds(s * ROWS_PER_TILE, ROWS_PER_TILE)],
        out_hbm.at[c, pl.ds(s * ROWS_PER_TILE, ROWS_PER_TILE)],
    )


@functools.cache
def _sc_kernels():
    mesh = plsc.VectorSubcoreMesh(
        core_axis_name="c", subcore_axis_name="s", num_cores=NC, num_subcores=NS
    )
    deg = pl.kernel(
        _deg_sc_body,
        out_type=jax.ShapeDtypeStruct((NC, NPAD, DEGW), jnp.float32),
        mesh=mesh,
        scratch_types=[
            pltpu.VMEM((CHUNKS_PER_TILE, CHUNK), jnp.int32),
            pltpu.VMEM((CHUNK, DEGW), jnp.float32),
            pltpu.VMEM((CHUNK,), jnp.int32),
            pltpu.VMEM_SHARED((NPAD, DEGW), jnp.float32),
        ],
    )
    agg = pl.kernel(
        _agg_sc_body,
        out_type=jax.ShapeDtypeStruct((NC, NPAD, D), jnp.float32),
        mesh=mesh,
        scratch_types=[
            pltpu.VMEM((IDXBLK, CHUNK), jnp.int32),
            pltpu.VMEM((IDXBLK, CHUNK), jnp.int32),
            pltpu.VMEM((CHUNK, D), jnp.float32),
            pltpu.VMEM((CHUNK, D), jnp.float32),
            pltpu.VMEM((CHUNK,), jnp.int32),
            pltpu.VMEM((CHUNK,), jnp.int32),
            pltpu.VMEM((CHUNK,), jnp.int32),
            pltpu.VMEM_SHARED((NPAD, D), jnp.float32),
            pltpu.SemaphoreType.DMA,
            pltpu.SemaphoreType.DMA,
        ],
    )
    return deg, agg


def _prep_body(deg_ref, x_ref, w_ref, dinv_ref, y_ref):
    deg = deg_ref[0, :, 0:1] + deg_ref[1, :, 0:1] + 1.0
    dinv = lax.rsqrt(deg)
    dinv_ref[...] = dinv
    y_ref[...] = (
        jnp.dot(x_ref[...], w_ref[...], preferred_element_type=jnp.float32) * dinv
    )


def _mid_layer(p_ref, y_ref, dinv_ref, b_ref, g_ref, beta_ref):
    pre = (p_ref[0] + p_ref[1] + y_ref[...]) * dinv_ref[...] + b_ref[...]
    valid = lax.broadcasted_iota(jnp.int32, (NPAD, 1), 0) < N
    pre = jnp.where(valid, pre, 0.0)
    mu = jnp.sum(pre, axis=0, keepdims=True) * (1.0 / N)
    var = jnp.sum(pre * pre, axis=0, keepdims=True) * (1.0 / N) - mu * mu
    h = (pre - mu) * lax.rsqrt(var + 1e-5) * g_ref[...] + beta_ref[...]
    h = jnp.maximum(h, 0.0)
    return jnp.where(valid, h, 0.0), valid


def _bn_body(p_ref, y_ref, dinv_ref, b_ref, g_ref, beta_ref, w_ref, ynext_ref):
    h, _ = _mid_layer(p_ref, y_ref, dinv_ref, b_ref, g_ref, beta_ref)
    ynext_ref[...] = (
        jnp.dot(h, w_ref[...], preferred_element_type=jnp.float32) * dinv_ref[...]
    )


def _head_body(p_ref, y_ref, dinv_ref, b_ref, g_ref, beta_ref, batch_ref,
               l1w_ref, l1b_ref, l2w_ref, l2b_ref, out_ref):
    h, valid = _mid_layer(p_ref, y_ref, dinv_ref, b_ref, g_ref, beta_ref)
    onehot = (
        batch_ref[...] == lax.broadcasted_iota(jnp.int32, (NPAD, G), 1)
    ).astype(jnp.float32)
    validf = valid.astype(jnp.float32)
    onehot = onehot * validf
    sums = lax.dot_general(
        onehot, h, (((0,), (0,)), ((), ())), preferred_element_type=jnp.float32
    )
    cnts = lax.dot_general(
        onehot, validf, (((0,), (0,)), ((), ())), preferred_element_type=jnp.float32
    )
    pooled = sums / jnp.maximum(cnts, 1.0)
    o = jnp.dot(pooled, l1w_ref[...], preferred_element_type=jnp.float32) + l1b_ref[...]
    o = jnp.maximum(o, 0.0)
    o = jnp.dot(o, l2w_ref[...], preferred_element_type=jnp.float32) + l2b_ref[...]
    m = jnp.max(o, axis=1, keepdims=True)
    lse = jnp.log(jnp.sum(jnp.exp(o - m), axis=1, keepdims=True)) + m
    out_ref[...] = o - lse


_prep_call = pl.pallas_call(
    _prep_body,
    out_shape=[
        jax.ShapeDtypeStruct((NPAD, 1), jnp.float32),
        jax.ShapeDtypeStruct((NPAD, D), jnp.float32),
    ],
)

_bn_call = pl.pallas_call(
    _bn_body,
    out_shape=jax.ShapeDtypeStruct((NPAD, D), jnp.float32),
)

_head_call = pl.pallas_call(
    _head_body,
    out_shape=jax.ShapeDtypeStruct((G, C), jnp.float32),
)


def kernel(x, edge_index, batch, W1, b1, g1, beta1, W2, b2, g2, beta2,
           W3, b3, g3, beta3, lin1_W, lin1_b, lin2_W, lin2_b):
    f32 = jnp.float32
    pad_e = EPAD - E
    src = jnp.concatenate(
        [edge_index[0], jnp.full((pad_e,), NPAD - 1, jnp.int32)]
    ).reshape(EPAD // CHUNK, CHUNK)
    dst = jnp.concatenate(
        [edge_index[1], jnp.full((pad_e,), NPAD - 1, jnp.int32)]
    ).reshape(EPAD // CHUNK, CHUNK)
    x_pad = jnp.zeros((NPAD, D), f32).at[:N].set(x)
    batch_pad = jnp.zeros((NPAD, 1), jnp.int32).at[:N, 0].set(batch)

    ones_rows = jnp.ones((CHUNK, DEGW), f32)
    zeros_deg = jnp.zeros((ROWS_PER_TILE, DEGW), f32)
    zeros_agg = jnp.zeros((ROWS_PER_TILE, D), f32)

    b1r, g1r, be1 = b1.reshape(1, D), g1.reshape(1, D), beta1.reshape(1, D)
    b2r, g2r, be2 = b2.reshape(1, D), g2.reshape(1, D), beta2.reshape(1, D)
    b3r, g3r, be3 = b3.reshape(1, D), g3.reshape(1, D), beta3.reshape(1, D)
    l1b = lin1_b.reshape(1, D)
    l2b = lin2_b.reshape(1, C)

    _deg_sc, _agg_sc = _sc_kernels()
    deg_parts = _deg_sc(dst, ones_rows, zeros_deg)
    dinv, y1 = _prep_call(deg_parts, x_pad, W1)

    p1 = _agg_sc(y1, src, dst, zeros_agg)
    y2 = _bn_call(p1, y1, dinv, b1r, g1r, be1, W2)

    p2 = _agg_sc(y2, src, dst, zeros_agg)
    y3 = _bn_call(p2, y2, dinv, b2r, g2r, be2, W3)

    p3 = _agg_sc(y3, src, dst, zeros_agg)
    out = _head_call(p3, y3, dinv, b3r, g3r, be3, batch_pad,
                     lin1_W, l1b, lin2_W, l2b)
    return out


# speedup vs baseline: 8.2991x; 1.0001x over previous
import functools

import jax
import jax.numpy as jnp
from jax import lax
from jax.experimental import pallas as pl
from jax.experimental.pallas import tpu as pltpu
from jax.experimental.pallas import tpu_sc as plsc

N = 10000
NPAD = 10240
E = 320000
D = 128
G = 128
C = 10

NC = 2
NS = 16
CHUNK = 128
EPAD = 327680
CHUNKS_PER_TILE = EPAD // (NC * NS * CHUNK)
SUBCHUNK = 64
SPT0 = 256
SPT1 = 64
SUBS_PER_BLK = 64
IDXROWS = SUBS_PER_BLK // 2
ROWS_PER_TILE = NPAD // NS
DEGW = 128

def _copy_idx_row(src2d, j, dst1d):
    for t in range(CHUNK // 16):
        dst1d[pl.ds(t * 16, 16)] = src2d[j, pl.ds(t * 16, 16)]


def _deg_sc_body(dst_hbm, ones_hbm, zeros_hbm, out_hbm, dst_v, ones_v, idx1,
                 accum):
    c = lax.axis_index("c")
    s = lax.axis_index("s")
    wid = s * NC + c
    pltpu.sync_copy(zeros_hbm, accum.at[pl.ds(s * ROWS_PER_TILE, ROWS_PER_TILE)])
    pltpu.sync_copy(ones_hbm, ones_v)
    pltpu.sync_copy(dst_hbm.at[pl.ds(wid * CHUNKS_PER_TILE, CHUNKS_PER_TILE)], dst_v)
    plsc.subcore_barrier()

    def body(j, carry):
        _copy_idx_row(dst_v, j, idx1)
        pltpu.sync_copy(ones_v, accum.at[idx1], add=True)
        return carry

    lax.fori_loop(0, CHUNKS_PER_TILE, body, 0)
    plsc.subcore_barrier()
    pltpu.sync_copy(
        accum.at[pl.ds(s * ROWS_PER_TILE, ROWS_PER_TILE)],
        out_hbm.at[c, pl.ds(s * ROWS_PER_TILE, ROWS_PER_TILE)],
    )


def _copy_idx_half(src2d, row, half, dst1d):
    for t in range(SUBCHUNK // 16):
        dst1d[pl.ds(t * 16, 16)] = src2d[row, pl.ds(half * SUBCHUNK + t * 16, 16)]


def _agg_sc_body(y_hbm, src_hbm, dst_hbm, zeros_hbm, out_hbm, src_v, dst_v,
                 r0, r1, r2, r3, g0, g1, g2, g3, d0, d1, d2, d3, accum,
                 gs0, gs1, gs2, gs3, ss0, ss1, ss2, ss3):
    rows = [r0, r1, r2, r3]
    gidx = [g0, g1, g2, g3]
    didx = [d0, d1, d2, d3]
    gsems = [gs0, gs1, gs2, gs3]
    ssems = [ss0, ss1, ss2, ss3]
    c = lax.axis_index("c")
    s = lax.axis_index("s")
    pltpu.sync_copy(zeros_hbm, accum.at[pl.ds(s * ROWS_PER_TILE, ROWS_PER_TILE)])
    plsc.subcore_barrier()

    nblk = jnp.where(c == 0, SPT0 // SUBS_PER_BLK, SPT1 // SUBS_PER_BLK)
    row_base0 = jnp.where(c == 0, s * SPT0, NS * SPT0 + s * SPT1) // 2

    for blk in range(SPT0 // SUBS_PER_BLK):
        @pl.when(blk < nblk)
        def _():
            rbase = pl.multiple_of(row_base0 + blk * IDXROWS, 8)
            pltpu.sync_copy(src_hbm.at[pl.ds(rbase, IDXROWS)], src_v)
            pltpu.sync_copy(dst_hbm.at[pl.ds(rbase, IDXROWS)], dst_v)

            def group(g, carry):
                j0 = 4 * g
                for u in range(4):
                    @pl.when(g > 0)
                    def _():
                        pltpu.make_async_copy(
                            rows[u], accum.at[didx[u]], ssems[u]
                        ).wait()
                    j = j0 + u
                    _copy_idx_half(src_v, j // 2, j % 2, gidx[u])
                    pltpu.async_copy(y_hbm.at[gidx[u]], rows[u],
                                     gsems[u])
                for u in range(4):
                    j = j0 + u
                    pltpu.make_async_copy(
                        y_hbm.at[gidx[u]], rows[u], gsems[u]
                    ).wait()
                    _copy_idx_half(dst_v, j // 2, j % 2, didx[u])
                    pltpu.async_copy(rows[u], accum.at[didx[u]],
                                     ssems[u], add=True)
                return carry

            lax.fori_loop(0, SUBS_PER_BLK // 4, group, 0)
            for u in range(4):
                pltpu.make_async_copy(
                    rows[u], accum.at[didx[u]], ssems[u]
                ).wait()
    plsc.subcore_barrier()
    pltpu.sync_copy(
        accum.at[pl.ds(s * ROWS_PER_TILE, ROWS_PER_TILE)],
        out_hbm.at[c, pl.ds(s * ROWS_PER_TILE, ROWS_PER_TILE)],
    )


@functools.cache
def _sc_kernels():
    mesh = plsc.VectorSubcoreMesh(
        core_axis_name="c", subcore_axis_name="s", num_cores=NC, num_subcores=NS
    )
    deg = pl.kernel(
        _deg_sc_body,
        out_type=jax.ShapeDtypeStruct((NC, NPAD, DEGW), jnp.float32),
        mesh=mesh,
        scratch_types=[
            pltpu.VMEM((CHUNKS_PER_TILE, CHUNK), jnp.int32),
            pltpu.VMEM((CHUNK, DEGW), jnp.float32),
            pltpu.VMEM((CHUNK,), jnp.int32),
            pltpu.VMEM_SHARED((NPAD, DEGW), jnp.float32),
        ],
    )
    agg = pl.kernel(
        _agg_sc_body,
        out_type=jax.ShapeDtypeStruct((NC, NPAD, D), jnp.float32),
        mesh=mesh,
        scratch_types=(
            [
                pltpu.VMEM((IDXROWS, CHUNK), jnp.int32),
                pltpu.VMEM((IDXROWS, CHUNK), jnp.int32),
            ]
            + [pltpu.VMEM((SUBCHUNK, D), jnp.float32) for _ in range(4)]
            + [pltpu.VMEM((SUBCHUNK,), jnp.int32) for _ in range(8)]
            + [
                pltpu.VMEM_SHARED((NPAD, D), jnp.float32),
            ]
            + [pltpu.SemaphoreType.DMA for _ in range(8)]
        ),
    )
    return deg, agg


def _prep_body(deg_ref, x_ref, w_ref, dinv_ref, y_ref):
    deg = deg_ref[0, :, 0:1] + deg_ref[1, :, 0:1] + 1.0
    dinv = lax.rsqrt(deg)
    dinv_ref[...] = dinv
    y_ref[...] = (
        jnp.dot(x_ref[...], w_ref[...], preferred_element_type=jnp.float32) * dinv
    )


def _mid_layer(p_ref, y_ref, dinv_ref, b_ref, g_ref, beta_ref):
    pre = (p_ref[0] + p_ref[1] + y_ref[...]) * dinv_ref[...] + b_ref[...]
    valid = lax.broadcasted_iota(jnp.int32, (NPAD, 1), 0) < N
    pre = jnp.where(valid, pre, 0.0)
    mu = jnp.sum(pre, axis=0, keepdims=True) * (1.0 / N)
    var = jnp.sum(pre * pre, axis=0, keepdims=True) * (1.0 / N) - mu * mu
    h = (pre - mu) * lax.rsqrt(var + 1e-5) * g_ref[...] + beta_ref[...]
    h = jnp.maximum(h, 0.0)
    return jnp.where(valid, h, 0.0), valid


def _bn_body(p_ref, y_ref, dinv_ref, b_ref, g_ref, beta_ref, w_ref, ynext_ref):
    h, _ = _mid_layer(p_ref, y_ref, dinv_ref, b_ref, g_ref, beta_ref)
    ynext_ref[...] = (
        jnp.dot(h, w_ref[...], preferred_element_type=jnp.float32) * dinv_ref[...]
    )


def _head_body(p_ref, y_ref, dinv_ref, b_ref, g_ref, beta_ref, batch_ref,
               l1w_ref, l1b_ref, l2w_ref, l2b_ref, out_ref):
    h, valid = _mid_layer(p_ref, y_ref, dinv_ref, b_ref, g_ref, beta_ref)
    onehot = (
        batch_ref[...] == lax.broadcasted_iota(jnp.int32, (NPAD, G), 1)
    ).astype(jnp.float32)
    validf = valid.astype(jnp.float32)
    onehot = onehot * validf
    sums = lax.dot_general(
        onehot, h, (((0,), (0,)), ((), ())), preferred_element_type=jnp.float32
    )
    cnts = lax.dot_general(
        onehot, validf, (((0,), (0,)), ((), ())), preferred_element_type=jnp.float32
    )
    pooled = sums / jnp.maximum(cnts, 1.0)
    o = jnp.dot(pooled, l1w_ref[...], preferred_element_type=jnp.float32) + l1b_ref[...]
    o = jnp.maximum(o, 0.0)
    o = jnp.dot(o, l2w_ref[...], preferred_element_type=jnp.float32) + l2b_ref[...]
    m = jnp.max(o, axis=1, keepdims=True)
    lse = jnp.log(jnp.sum(jnp.exp(o - m), axis=1, keepdims=True)) + m
    out_ref[...] = o - lse


_prep_call = pl.pallas_call(
    _prep_body,
    out_shape=[
        jax.ShapeDtypeStruct((NPAD, 1), jnp.float32),
        jax.ShapeDtypeStruct((NPAD, D), jnp.float32),
    ],
)

_bn_call = pl.pallas_call(
    _bn_body,
    out_shape=jax.ShapeDtypeStruct((NPAD, D), jnp.float32),
)

_head_call = pl.pallas_call(
    _head_body,
    out_shape=jax.ShapeDtypeStruct((G, C), jnp.float32),
)


def kernel(x, edge_index, batch, W1, b1, g1, beta1, W2, b2, g2, beta2,
           W3, b3, g3, beta3, lin1_W, lin1_b, lin2_W, lin2_b):
    f32 = jnp.float32
    pad_e = EPAD - E
    src = jnp.concatenate(
        [edge_index[0], jnp.full((pad_e,), NPAD - 1, jnp.int32)]
    ).reshape(EPAD // CHUNK, CHUNK)
    dst = jnp.concatenate(
        [edge_index[1], jnp.full((pad_e,), NPAD - 1, jnp.int32)]
    ).reshape(EPAD // CHUNK, CHUNK)
    x_pad = jnp.zeros((NPAD, D), f32).at[:N].set(x)
    batch_pad = jnp.zeros((NPAD, 1), jnp.int32).at[:N, 0].set(batch)

    ones_rows = jnp.ones((CHUNK, DEGW), f32)
    zeros_deg = jnp.zeros((ROWS_PER_TILE, DEGW), f32)
    zeros_agg = jnp.zeros((ROWS_PER_TILE, D), f32)

    b1r, g1r, be1 = b1.reshape(1, D), g1.reshape(1, D), beta1.reshape(1, D)
    b2r, g2r, be2 = b2.reshape(1, D), g2.reshape(1, D), beta2.reshape(1, D)
    b3r, g3r, be3 = b3.reshape(1, D), g3.reshape(1, D), beta3.reshape(1, D)
    l1b = lin1_b.reshape(1, D)
    l2b = lin2_b.reshape(1, C)

    _deg_sc, _agg_sc = _sc_kernels()
    deg_parts = _deg_sc(dst, ones_rows, zeros_deg)
    dinv, y1 = _prep_call(deg_parts, x_pad, W1)

    p1 = _agg_sc(y1, src, dst, zeros_agg)
    y2 = _bn_call(p1, y1, dinv, b1r, g1r, be1, W2)

    p2 = _agg_sc(y2, src, dst, zeros_agg)
    y3 = _bn_call(p2, y2, dinv, b2r, g2r, be2, W3)

    p3 = _agg_sc(y3, src, dst, zeros_agg)
    out = _head_call(p3, y3, dinv, b3r, g3r, be3, batch_pad,
                     lin1_W, l1b, lin2_W, l2b)
    return out
